# baseline (speedup 1.0000x reference)
import jax
import jax.numpy as jnp
from jax.experimental import pallas as pl
from jax.experimental.pallas import tpu as pltpu

_N = 8192
_R = 512
_W = 512
_NCH = _N // _W
_K_NEAR = 8
_K_FAR = 16
_TEMP_LOC = 0.1
_TEMP_CON = 0.5
_INF = jnp.inf


def _block_body(lo_ref, hi_ref, sc_col, bid_col, crd_blk,
                sc_ch, bid_ch, crd_ch,
                o_wsd, o_wsum, o_lpos, o_lneg, o_dtot, o_dcnt, o_smooth,
                d2_ref, mbuf_ref, sbuf_ref):
    i = pl.program_id(0)
    cs = lo_ref[i] // _W
    ce = (hi_ref[i] + _W - 1) // _W

    row_sc = sc_col[...]
    row_b = bid_col[...]
    cb = crd_blk[...]
    ridx = i * _R + jax.lax.broadcasted_iota(jnp.int32, (_R, 1), 0)
    iota_w = jax.lax.broadcasted_iota(jnp.int32, (_R, _W), 1)
    zero_r1 = jnp.zeros((_R, 1), jnp.float32)

    def build(c, carry):
        rk, nb = carry
        a_sc = sc_ch[c]
        a_b = bid_ch[c]
        ct = crd_ch[c]
        d2c = ((cb[:, 0:1] - ct[0:1, :]) ** 2
               + (cb[:, 1:2] - ct[1:2, :]) ** 2
               + (cb[:, 2:3] - ct[2:3, :]) ** 2)
        same = row_b == a_b
        d2_ref[c] = jnp.where(same, d2c, _INF)
        jidx = c * _W + iota_w
        less = (a_sc < row_sc) | ((a_sc == row_sc) & (jidx < ridx))
        rk = rk + jnp.sum(jnp.where(same & less, 1.0, 0.0),
                          axis=1, keepdims=True)
        nb = nb + jnp.sum(jnp.where(same, 1.0, 0.0), axis=1, keepdims=True)
        return rk, nb

    rank, n_b = jax.lax.fori_loop(cs, ce, build, (zero_r1, zero_r1))

    inf_w = jnp.full((_R, _W), _INF, jnp.float32)
    d2_ref[ce] = inf_w
    d2_ref[ce + 1] = inf_w
    d2_ref[ce + 2] = inf_w

    step = 1.0 / (n_b - 1.0)
    sqe = (row_sc - rank * step) ** 2
    ok = n_b >= 2.0
    dtot = jnp.sum(jnp.where(ok, sqe / n_b, 0.0), axis=(0, 1), keepdims=True)
    dcnt = jnp.sum(jnp.where(ok, 1.0 / n_b, 0.0), axis=(0, 1), keepdims=True)

    zero11 = jnp.zeros((1, 1), jnp.float32)
    def lexlt(a, b):
        return (a[0] < b[0]) | ((a[0] == b[0]) & (a[1] < b[1]))

    def sel3(t, a, b):
        return (jnp.where(t, a[0], b[0]), jnp.where(t, a[1], b[1]),
                jnp.where(t, a[2], b[2]))

    def merge2(x, y):
        x1, x2 = x
        y1, y2 = y
        t = lexlt(x1, y1)
        first = sel3(t, x1, y1)
        loser = sel3(t, y1, x1)
        nxt = sel3(t, x2, y2)
        second = sel3(lexlt(loser, nxt), loser, nxt)
        return first, second

    prev_m = jnp.full((_R, 1), -_INF, jnp.float32)
    prev_gi = jnp.full((_R, 1), -1, jnp.int32)
    ngroups = (ce - cs + 3) // 4
    for k in range(_K_FAR // 2):
        def chunk_min2(cc, prev_m=prev_m, prev_gi=prev_gi):
            d2c = d2_ref[cc]
            gidx = cc * _W + iota_w
            scb = sc_ch[jnp.minimum(cc, _NCH - 1)]
            cand = (d2c > prev_m) | ((d2c == prev_m) & (gidx > prev_gi))
            d2m = jnp.where(cand, d2c, _INF)
            mc1 = jnp.min(d2m, axis=1, keepdims=True)
            idx1 = jnp.min(jnp.where(d2m == mc1, gidx, _N), axis=1,
                           keepdims=True)
            sel1 = gidx == idx1
            sc1 = jnp.sum(jnp.where(sel1, scb, 0.0), axis=1, keepdims=True)
            d2m2 = jnp.where(sel1, _INF, d2m)
            mc2 = jnp.min(d2m2, axis=1, keepdims=True)
            idx2 = jnp.min(jnp.where(d2m2 == mc2, gidx, _N), axis=1,
                           keepdims=True)
            sc2 = jnp.sum(jnp.where(gidx == idx2, scb, 0.0), axis=1,
                          keepdims=True)
            return (mc1, idx1, sc1), (mc2, idx2, sc2)

        def sweep(t, carry):
            c0 = cs + 4 * t
            r = merge2(merge2(chunk_min2(c0), chunk_min2(c0 + 1)),
                       merge2(chunk_min2(c0 + 2), chunk_min2(c0 + 3)))
            return merge2(carry, r)

        infty = (jnp.full((_R, 1), _INF, jnp.float32),
                 jnp.full((_R, 1), _N, jnp.int32), zero_r1)
        (m1, g1, s1), (m2, g2, s2) = jax.lax.fori_loop(
            0, ngroups, sweep, (infty, infty))
        prev_m, prev_gi = m2, g2
        mbuf_ref[:, 2 * k:2 * k + 1] = m1
        sbuf_ref[:, 2 * k:2 * k + 1] = s1
        mbuf_ref[:, 2 * k + 1:2 * k + 2] = m2
        sbuf_ref[:, 2 * k + 1:2 * k + 2] = s2

    mb = mbuf_ref[...]
    sb = sbuf_ref[...]
    mn = mb[:, :_K_NEAR]
    sn = sb[:, :_K_NEAR]
    sf = sb[:, _K_NEAR:]
    sd = row_sc - sn
    w = jnp.exp(-jnp.sqrt(jnp.maximum(mn, 0.0)) / _TEMP_LOC)
    wsum = jnp.sum(w, axis=(0, 1), keepdims=True)
    wsd = jnp.sum(w * sd * sd, axis=(0, 1), keepdims=True)
    sim = 1.0 - jnp.abs(sd)
    lpos = jnp.sum(jnp.log(jax.nn.sigmoid(sim / _TEMP_CON) + 1e-8),
                   axis=(0, 1), keepdims=True)
    simf = 1.0 - jnp.abs(row_sc - sf)
    lneg = jnp.sum(jnp.log(1.0 - jax.nn.sigmoid(simf / _TEMP_CON) + 1e-8),
                   axis=(0, 1), keepdims=True)
    near_sum = jnp.sum(sn, axis=1, keepdims=True)
    smooth = jnp.sum((row_sc - near_sum * (1.0 / _K_NEAR)) ** 2,
                     axis=(0, 1), keepdims=True)

    @pl.when(i == 0)
    def _init():
        o_wsd[...] = zero11
        o_wsum[...] = zero11
        o_lpos[...] = zero11
        o_lneg[...] = zero11
        o_dtot[...] = zero11
        o_dcnt[...] = zero11
        o_smooth[...] = zero11

    o_wsd[...] += wsd
    o_wsum[...] += wsum
    o_lpos[...] += lpos
    o_lneg[...] += lneg
    o_dtot[...] += dtot
    o_dcnt[...] += dcnt
    o_smooth[...] += smooth


def _run(scores, coords, batch_ids):
    n = scores.shape[0]
    num_blocks = n // _R
    ids2 = batch_ids.reshape(num_blocks, _R)
    lo = jnp.searchsorted(batch_ids, ids2[:, 0], side='left').astype(
        jnp.int32)
    hi = jnp.searchsorted(batch_ids, ids2[:, -1], side='right').astype(
        jnp.int32)

    sc_col = scores.reshape(n, 1)
    bid_col = batch_ids.reshape(n, 1)
    sc_ch = scores.reshape(_NCH, 1, _W)
    bid_ch = batch_ids.reshape(_NCH, 1, _W)
    crd_ch = coords.reshape(_NCH, _W, 3).transpose(0, 2, 1)

    one = pl.BlockSpec((1, 1), lambda i, *_: (0, 0))
    grid_spec = pltpu.PrefetchScalarGridSpec(
        num_scalar_prefetch=2,
        grid=(num_blocks,),
        in_specs=[
            pl.BlockSpec((_R, 1), lambda i, *_: (i, 0)),
            pl.BlockSpec((_R, 1), lambda i, *_: (i, 0)),
            pl.BlockSpec((_R, 3), lambda i, *_: (i, 0)),
            pl.BlockSpec((_NCH, 1, _W), lambda i, *_: (0, 0, 0)),
            pl.BlockSpec((_NCH, 1, _W), lambda i, *_: (0, 0, 0)),
            pl.BlockSpec((_NCH, 3, _W), lambda i, *_: (0, 0, 0)),
        ],
        out_specs=[one] * 7,
        scratch_shapes=[pltpu.VMEM((_NCH + 3, _R, _W), jnp.float32),
                        pltpu.VMEM((_R, _K_FAR), jnp.float32),
                        pltpu.VMEM((_R, _K_FAR), jnp.float32)],
    )
    outs = pl.pallas_call(
        _block_body,
        grid_spec=grid_spec,
        out_shape=[jax.ShapeDtypeStruct((1, 1), jnp.float32)] * 7,
    )(lo, hi, sc_col, bid_col, coords, sc_ch, bid_ch, crd_ch)
    return outs


@jax.jit
def kernel(scores, coords, batch_ids):
    n = scores.shape[0]
    wsd, wsum, lpos, lneg, dtot, dcnt, smooth = [o[0, 0] for o in
                                                 _run(scores, coords,
                                                      batch_ids)]
    loss_loc = wsd / jnp.maximum(wsum, 1e-8)
    loss_con = -(lpos + lneg) / (n * _K_NEAR)
    loss_dist = dtot / jnp.maximum(jnp.round(dcnt), 1.0)
    loss_smooth = smooth / n
    return (1.0 * loss_loc + 0.5 * loss_con
            + 0.3 * loss_dist + 0.2 * loss_smooth)


# speedup vs baseline: 1.4763x; 1.4763x over previous
import jax
import jax.numpy as jnp
from jax.experimental import pallas as pl
from jax.experimental.pallas import tpu as pltpu

_N = 8192
_R = 256
_W = 1024
_NCH = _N // _W
_K_NEAR = 8
_K_FAR = 16
_TEMP_LOC = 0.1
_TEMP_CON = 0.5
_INF = jnp.inf


def _block_body(lo_ref, hi_ref, sc_col, bid_col, crd_blk,
                sc_ch, bid_ch, crd_ch,
                o_wsd, o_wsum, o_lpos, o_lneg, o_dtot, o_dcnt, o_smooth,
                d2_ref, mbuf_ref, sbuf_ref):
    i = pl.program_id(0)
    cs = lo_ref[i] // _W
    ce = (hi_ref[i] + _W - 1) // _W

    row_sc = sc_col[...]
    row_b = bid_col[...]
    cb = crd_blk[...]
    ridx = i * _R + jax.lax.broadcasted_iota(jnp.int32, (_R, 1), 0)
    iota_w = jax.lax.broadcasted_iota(jnp.int32, (_R, _W), 1)
    zero_r1 = jnp.zeros((_R, 1), jnp.float32)

    def build(c, carry):
        rk, nb = carry
        a_sc = sc_ch[c]
        a_b = bid_ch[c]
        ct = crd_ch[c]
        d2c = ((cb[:, 0:1] - ct[0:1, :]) ** 2
               + (cb[:, 1:2] - ct[1:2, :]) ** 2
               + (cb[:, 2:3] - ct[2:3, :]) ** 2)
        same = row_b == a_b
        d2_ref[c] = jnp.where(same, d2c, _INF)
        jidx = c * _W + iota_w
        less = (a_sc < row_sc) | ((a_sc == row_sc) & (jidx < ridx))
        rk = rk + jnp.sum(jnp.where(same & less, 1.0, 0.0),
                          axis=1, keepdims=True)
        nb = nb + jnp.sum(jnp.where(same, 1.0, 0.0), axis=1, keepdims=True)
        return rk, nb

    rank, n_b = jax.lax.fori_loop(cs, ce, build, (zero_r1, zero_r1))

    d2_ref[ce] = jnp.full((_R, _W), _INF, jnp.float32)

    step = 1.0 / (n_b - 1.0)
    sqe = (row_sc - rank * step) ** 2
    ok = n_b >= 2.0
    dtot = jnp.sum(jnp.where(ok, sqe / n_b, 0.0), axis=(0, 1), keepdims=True)
    dcnt = jnp.sum(jnp.where(ok, 1.0 / n_b, 0.0), axis=(0, 1), keepdims=True)

    zero11 = jnp.zeros((1, 1), jnp.float32)
    def lexlt(a, b):
        return (a[0] < b[0]) | ((a[0] == b[0]) & (a[1] < b[1]))

    def sel3(t, a, b):
        return (jnp.where(t, a[0], b[0]), jnp.where(t, a[1], b[1]),
                jnp.where(t, a[2], b[2]))

    def merge2(x, y):
        x1, x2 = x
        y1, y2 = y
        t = lexlt(x1, y1)
        first = sel3(t, x1, y1)
        loser = sel3(t, y1, x1)
        nxt = sel3(t, x2, y2)
        second = sel3(lexlt(loser, nxt), loser, nxt)
        return first, second

    prev_m = jnp.full((_R, 1), -_INF, jnp.float32)
    prev_gi = jnp.full((_R, 1), -1, jnp.int32)
    ngroups = (ce - cs + 1) // 2
    for k in range(_K_FAR // 2):
        def chunk_min2(cc, prev_m=prev_m, prev_gi=prev_gi):
            d2c = d2_ref[cc]
            gidx = cc * _W + iota_w
            scb = sc_ch[jnp.minimum(cc, _NCH - 1)]
            cand = (d2c > prev_m) | ((d2c == prev_m) & (gidx > prev_gi))
            d2m = jnp.where(cand, d2c, _INF)
            mc1 = jnp.min(d2m, axis=1, keepdims=True)
            idx1 = jnp.min(jnp.where(d2m == mc1, gidx, _N), axis=1,
                           keepdims=True)
            sel1 = gidx == idx1
            sc1 = jnp.sum(jnp.where(sel1, scb, 0.0), axis=1, keepdims=True)
            d2m2 = jnp.where(sel1, _INF, d2m)
            mc2 = jnp.min(d2m2, axis=1, keepdims=True)
            idx2 = jnp.min(jnp.where(d2m2 == mc2, gidx, _N), axis=1,
                           keepdims=True)
            sc2 = jnp.sum(jnp.where(gidx == idx2, scb, 0.0), axis=1,
                          keepdims=True)
            return (mc1, idx1, sc1), (mc2, idx2, sc2)

        def sweep(t, carry):
            c0 = cs + 2 * t
            r = merge2(chunk_min2(c0), chunk_min2(c0 + 1))
            return merge2(carry, r)

        infty = (jnp.full((_R, 1), _INF, jnp.float32),
                 jnp.full((_R, 1), _N, jnp.int32), zero_r1)
        (m1, g1, s1), (m2, g2, s2) = jax.lax.fori_loop(
            0, ngroups, sweep, (infty, infty))
        prev_m, prev_gi = m2, g2
        mbuf_ref[:, 2 * k:2 * k + 1] = m1
        sbuf_ref[:, 2 * k:2 * k + 1] = s1
        mbuf_ref[:, 2 * k + 1:2 * k + 2] = m2
        sbuf_ref[:, 2 * k + 1:2 * k + 2] = s2

    mb = mbuf_ref[...]
    sb = sbuf_ref[...]
    mn = mb[:, :_K_NEAR]
    sn = sb[:, :_K_NEAR]
    sf = sb[:, _K_NEAR:]
    sd = row_sc - sn
    w = jnp.exp(-jnp.sqrt(jnp.maximum(mn, 0.0)) / _TEMP_LOC)
    wsum = jnp.sum(w, axis=(0, 1), keepdims=True)
    wsd = jnp.sum(w * sd * sd, axis=(0, 1), keepdims=True)
    sim = 1.0 - jnp.abs(sd)
    lpos = jnp.sum(jnp.log(jax.nn.sigmoid(sim / _TEMP_CON) + 1e-8),
                   axis=(0, 1), keepdims=True)
    simf = 1.0 - jnp.abs(row_sc - sf)
    lneg = jnp.sum(jnp.log(1.0 - jax.nn.sigmoid(simf / _TEMP_CON) + 1e-8),
                   axis=(0, 1), keepdims=True)
    near_sum = jnp.sum(sn, axis=1, keepdims=True)
    smooth = jnp.sum((row_sc - near_sum * (1.0 / _K_NEAR)) ** 2,
                     axis=(0, 1), keepdims=True)

    @pl.when(i == 0)
    def _init():
        o_wsd[...] = zero11
        o_wsum[...] = zero11
        o_lpos[...] = zero11
        o_lneg[...] = zero11
        o_dtot[...] = zero11
        o_dcnt[...] = zero11
        o_smooth[...] = zero11

    o_wsd[...] += wsd
    o_wsum[...] += wsum
    o_lpos[...] += lpos
    o_lneg[...] += lneg
    o_dtot[...] += dtot
    o_dcnt[...] += dcnt
    o_smooth[...] += smooth


def _run(scores, coords, batch_ids):
    n = scores.shape[0]
    num_blocks = n // _R
    ids2 = batch_ids.reshape(num_blocks, _R)
    lo = jnp.searchsorted(batch_ids, ids2[:, 0], side='left').astype(
        jnp.int32)
    hi = jnp.searchsorted(batch_ids, ids2[:, -1], side='right').astype(
        jnp.int32)

    sc_col = scores.reshape(n, 1)
    bid_col = batch_ids.reshape(n, 1)
    sc_ch = scores.reshape(_NCH, 1, _W)
    bid_ch = batch_ids.reshape(_NCH, 1, _W)
    crd_ch = coords.reshape(_NCH, _W, 3).transpose(0, 2, 1)

    one = pl.BlockSpec((1, 1), lambda i, *_: (0, 0))
    grid_spec = pltpu.PrefetchScalarGridSpec(
        num_scalar_prefetch=2,
        grid=(num_blocks,),
        in_specs=[
            pl.BlockSpec((_R, 1), lambda i, *_: (i, 0)),
            pl.BlockSpec((_R, 1), lambda i, *_: (i, 0)),
            pl.BlockSpec((_R, 3), lambda i, *_: (i, 0)),
            pl.BlockSpec((_NCH, 1, _W), lambda i, *_: (0, 0, 0)),
            pl.BlockSpec((_NCH, 1, _W), lambda i, *_: (0, 0, 0)),
            pl.BlockSpec((_NCH, 3, _W), lambda i, *_: (0, 0, 0)),
        ],
        out_specs=[one] * 7,
        scratch_shapes=[pltpu.VMEM((_NCH + 1, _R, _W), jnp.float32),
                        pltpu.VMEM((_R, _K_FAR), jnp.float32),
                        pltpu.VMEM((_R, _K_FAR), jnp.float32)],
    )
    outs = pl.pallas_call(
        _block_body,
        grid_spec=grid_spec,
        out_shape=[jax.ShapeDtypeStruct((1, 1), jnp.float32)] * 7,
    )(lo, hi, sc_col, bid_col, coords, sc_ch, bid_ch, crd_ch)
    return outs


@jax.jit
def kernel(scores, coords, batch_ids):
    n = scores.shape[0]
    wsd, wsum, lpos, lneg, dtot, dcnt, smooth = [o[0, 0] for o in
                                                 _run(scores, coords,
                                                      batch_ids)]
    loss_loc = wsd / jnp.maximum(wsum, 1e-8)
    loss_con = -(lpos + lneg) / (n * _K_NEAR)
    loss_dist = dtot / jnp.maximum(jnp.round(dcnt), 1.0)
    loss_smooth = smooth / n
    return (1.0 * loss_loc + 0.5 * loss_con
            + 0.3 * loss_dist + 0.2 * loss_smooth)
